# baseline (speedup 1.0000x reference)
import jax
import jax.numpy as jnp
from jax.experimental import pallas as pl

_CHUNK = 4096


def _ohem_reduce_kernel(loc_p_ref, loc_t_ref, cls_ref, sum_ref, cnt_ref):
    g = pl.program_id(0)

    @pl.when(g == 0)
    def _init():
        sum_ref[...] = jnp.zeros_like(sum_ref)
        cnt_ref[...] = jnp.zeros_like(cnt_ref)

    diff = loc_p_ref[0] - loc_t_ref[0]
    absd = jnp.abs(diff)
    sl1 = jnp.where(absd < 1.0, 0.5 * diff * diff, absd - 0.5)

    m = (jnp.clip(cls_ref[0], 0, 1) > 0).astype(jnp.float32)
    m8 = jnp.repeat(m, 8, axis=1)

    sum_ref[...] += jnp.sum(sl1 * m8, keepdims=True)
    cnt_ref[...] += jnp.sum(m, keepdims=True)


def kernel(loc_preds, loc_targets, cls_preds, cls_targets):
    B, A, D = loc_preds.shape
    nchunks = A // _CHUNK
    G = B * nchunks
    lp = loc_preds.reshape(G, 1, _CHUNK * D)
    lt = loc_targets.reshape(G, 1, _CHUNK * D)
    ct = cls_targets.astype(jnp.int32).reshape(G, 1, _CHUNK)

    out_sum, out_cnt = pl.pallas_call(
        _ohem_reduce_kernel,
        grid=(G,),
        in_specs=[
            pl.BlockSpec((1, 1, _CHUNK * D), lambda g: (g, 0, 0)),
            pl.BlockSpec((1, 1, _CHUNK * D), lambda g: (g, 0, 0)),
            pl.BlockSpec((1, 1, _CHUNK), lambda g: (g, 0, 0)),
        ],
        out_specs=[
            pl.BlockSpec((1, 1), lambda g: (0, 0)),
            pl.BlockSpec((1, 1), lambda g: (0, 0)),
        ],
        out_shape=[
            jax.ShapeDtypeStruct((1, 1), jnp.float32),
            jax.ShapeDtypeStruct((1, 1), jnp.float32),
        ],
    )(lp, lt, ct)

    return (0.2 * out_sum[0, 0] / out_cnt[0, 0]).astype(jnp.float32)


# speedup vs baseline: 2.4901x; 2.4901x over previous
import jax
import jax.numpy as jnp
from jax.experimental import pallas as pl

_BS = 4096


def _ohem_reduce_kernel(loc_p_ref, loc_t_ref, cls_ref, sum_ref, cnt_ref):
    g = pl.program_id(0)

    @pl.when(g == 0)
    def _init():
        sum_ref[...] = jnp.zeros_like(sum_ref)
        cnt_ref[...] = jnp.zeros_like(cnt_ref)

    diff = loc_p_ref[...] - loc_t_ref[...]
    absd = jnp.abs(diff)
    sl1 = jnp.where(absd < 1.0, 0.5 * diff * diff, absd - 0.5)

    lane = jax.lax.broadcasted_iota(jnp.int32, (128, 16), 0)
    slot = jax.lax.broadcasted_iota(jnp.int32, (128, 16), 1)
    q = (lane // 8 == slot).astype(jnp.float32)

    per_anchor = jax.lax.dot_general(
        sl1, q, (((1,), (0,)), ((), ())),
        precision=jax.lax.Precision.HIGHEST,
        preferred_element_type=jnp.float32,
    )

    m = (jnp.clip(cls_ref[...], 0, 1) > 0).astype(jnp.float32)

    sum_ref[...] += jnp.sum(per_anchor * m, keepdims=True)
    cnt_ref[...] += jnp.sum(m, keepdims=True)


def kernel(loc_preds, loc_targets, cls_preds, cls_targets):
    B, A, D = loc_preds.shape
    rows = B * A * D // 128
    lp = loc_preds.reshape(rows, 128)
    lt = loc_targets.reshape(rows, 128)
    ct = cls_targets.astype(jnp.int32).reshape(rows, 16)

    grid = (rows // _BS,)

    out_sum, out_cnt = pl.pallas_call(
        _ohem_reduce_kernel,
        grid=grid,
        in_specs=[
            pl.BlockSpec((_BS, 128), lambda g: (g, 0)),
            pl.BlockSpec((_BS, 128), lambda g: (g, 0)),
            pl.BlockSpec((_BS, 16), lambda g: (g, 0)),
        ],
        out_specs=[
            pl.BlockSpec((1, 1), lambda g: (0, 0)),
            pl.BlockSpec((1, 1), lambda g: (0, 0)),
        ],
        out_shape=[
            jax.ShapeDtypeStruct((1, 1), jnp.float32),
            jax.ShapeDtypeStruct((1, 1), jnp.float32),
        ],
    )(lp, lt, ct)

    return (0.2 * out_sum[0, 0] / out_cnt[0, 0]).astype(jnp.float32)


# speedup vs baseline: 61.1773x; 24.5687x over previous
import jax
import jax.numpy as jnp
from jax.experimental import pallas as pl
from jax.experimental.pallas import tpu as pltpu

_NB = 4


def _ohem_reduce_kernel(loc_p_ref, loc_t_ref, cls_ref, sum_ref, cnt_ref):
    diff = loc_p_ref[...] - loc_t_ref[...]
    absd = jnp.abs(diff)
    t = jnp.minimum(absd, 1.0)
    sl1 = t * (absd - 0.5 * t)

    m = (jnp.clip(cls_ref[...], 0, 1) > 0).astype(jnp.float32)

    sum_ref[...] = jnp.sum(sl1 * m).reshape(1, 1, 1)
    cnt_ref[...] = jnp.sum(m).reshape(1, 1, 1)


def kernel(loc_preds, loc_targets, cls_preds, cls_targets):
    B, A, D = loc_preds.shape
    lp = jnp.transpose(loc_preds, (0, 2, 1))
    lt = jnp.transpose(loc_targets, (0, 2, 1))
    ct = cls_targets.astype(jnp.int32).reshape(B, 1, A)

    G = B // _NB

    out_sum, out_cnt = pl.pallas_call(
        _ohem_reduce_kernel,
        grid=(G,),
        in_specs=[
            pl.BlockSpec((_NB, D, A), lambda g: (g, 0, 0)),
            pl.BlockSpec((_NB, D, A), lambda g: (g, 0, 0)),
            pl.BlockSpec((_NB, 1, A), lambda g: (g, 0, 0)),
        ],
        out_specs=[
            pl.BlockSpec((1, 1, 1), lambda g: (g, 0, 0)),
            pl.BlockSpec((1, 1, 1), lambda g: (g, 0, 0)),
        ],
        out_shape=[
            jax.ShapeDtypeStruct((G, 1, 1), jnp.float32),
            jax.ShapeDtypeStruct((G, 1, 1), jnp.float32),
        ],
        compiler_params=pltpu.CompilerParams(
            dimension_semantics=("parallel",),
        ),
    )(lp, lt, ct)

    return (0.2 * jnp.sum(out_sum) / jnp.sum(out_cnt)).astype(jnp.float32)


# speedup vs baseline: 63.9604x; 1.0455x over previous
import jax
import jax.numpy as jnp
from jax.experimental import pallas as pl

_NB = 4


def _ohem_reduce_kernel(loc_p_ref, loc_t_ref, cls_ref, sum_ref, cnt_ref):
    g = pl.program_id(0)

    @pl.when(g == 0)
    def _init():
        sum_ref[...] = jnp.zeros_like(sum_ref)
        cnt_ref[...] = jnp.zeros_like(cnt_ref)

    diff = loc_p_ref[...] - loc_t_ref[...]
    absd = jnp.abs(diff)
    t = jnp.minimum(absd, 1.0)
    sl1 = t * (absd - 0.5 * t)

    m = (jnp.clip(cls_ref[...], 0, 1) > 0).astype(jnp.float32)

    sum_ref[...] += jnp.sum(sl1 * m).reshape(1, 1)
    cnt_ref[...] += jnp.sum(m).reshape(1, 1)


def kernel(loc_preds, loc_targets, cls_preds, cls_targets):
    B, A, D = loc_preds.shape
    lp = jnp.transpose(loc_preds, (0, 2, 1))
    lt = jnp.transpose(loc_targets, (0, 2, 1))
    ct = cls_targets.astype(jnp.int32).reshape(B, 1, A)

    grid = (B // _NB,)

    out_sum, out_cnt = pl.pallas_call(
        _ohem_reduce_kernel,
        grid=grid,
        in_specs=[
            pl.BlockSpec((_NB, D, A), lambda g: (g, 0, 0)),
            pl.BlockSpec((_NB, D, A), lambda g: (g, 0, 0)),
            pl.BlockSpec((_NB, 1, A), lambda g: (g, 0, 0)),
        ],
        out_specs=[
            pl.BlockSpec((1, 1), lambda g: (0, 0)),
            pl.BlockSpec((1, 1), lambda g: (0, 0)),
        ],
        out_shape=[
            jax.ShapeDtypeStruct((1, 1), jnp.float32),
            jax.ShapeDtypeStruct((1, 1), jnp.float32),
        ],
    )(lp, lt, ct)

    return (0.2 * out_sum[0, 0] / out_cnt[0, 0]).astype(jnp.float32)
